# Initial kernel scaffold; baseline (speedup 1.0000x reference)
import jax
import jax.numpy as jnp
from jax.experimental import pallas as pl


def kernel(x, edge_index, W0, b0, W1, b1, W2, b2, W3, b3, W4, b4, W5, b5, g1, be1, g2, be2):
    raise NotImplementedError("write your pallas kernel here")


# speedup vs baseline: 3.5772x; 3.5772x over previous
import functools

import jax
import jax.numpy as jnp
from jax import lax
from jax.experimental import pallas as pl
from jax.experimental.pallas import tpu as pltpu
from jax.experimental.pallas import tpu_sc as plsc

F32 = jnp.float32
CHUNK = 128
ZR = 784
R = 512
_SC_PARAMS = pltpu.CompilerParams(use_tc_tiling_on_sc=False)


def _pads(n, e):
    n_pad = ((n + 25087) // 25088) * 25088
    e_pad = ((e + 4095) // 4096) * 4096
    return n_pad, e_pad


@functools.lru_cache(maxsize=None)
def _make_agg(n_pad, e_pad):
    tr = n_pad // 16
    zch = tr // ZR
    ch = e_pad // (2 * 16 * CHUNK)
    mesh = plsc.VectorSubcoreMesh(core_axis_name="c", subcore_axis_name="s")

    @functools.partial(
        pl.kernel,
        mesh=mesh,
        out_type=jax.ShapeDtypeStruct((2, n_pad, 16), F32),
        compiler_params=_SC_PARAMS,
        scratch_types=[
            pltpu.VMEM((CHUNK,), jnp.int32),
            pltpu.VMEM((CHUNK,), jnp.int32),
            pltpu.VMEM((CHUNK, 16), F32),
            pltpu.VMEM((ZR, 16), F32),
            pltpu.VMEM_SHARED((n_pad, 16), F32),
            pltpu.SemaphoreType.DMA,
        ],
    )
    def agg(s_hbm, src_hbm, dst_hbm, out_hbm, sidx, didx, rows, zbuf, acc, sem):
        c = lax.axis_index("c")
        t = lax.axis_index("s")
        row0 = t * tr

        def _zfill(r, carry):
            zbuf[r, pl.ds(0, 16)] = jnp.zeros((16,), F32)
            return carry

        lax.fori_loop(0, ZR, _zfill, 0)

        for z in range(zch):
            pltpu.sync_copy(zbuf, acc.at[pl.ds(row0 + z * ZR, ZR)])
        plsc.subcore_barrier()

        ebase = c * (e_pad // 2) + t * (ch * CHUNK)

        def _edges(k, carry):
            off = ebase + k * CHUNK
            pltpu.sync_copy(src_hbm.at[pl.ds(off, CHUNK)], sidx)
            pltpu.async_copy(s_hbm.at[sidx], rows, sem).wait()
            pltpu.sync_copy(dst_hbm.at[pl.ds(off, CHUNK)], didx)
            pltpu.sync_copy(rows, acc.at[didx], add=True)
            return carry

        lax.fori_loop(0, ch, _edges, 0)
        plsc.subcore_barrier()
        for z in range(zch):
            r0 = row0 + z * ZR
            pltpu.sync_copy(acc.at[pl.ds(r0, ZR)], out_hbm.at[c, pl.ds(r0, ZR)])
        plsc.subcore_barrier()

    return agg


@functools.lru_cache(maxsize=None)
def _make_deg(n_pad, e_pad):
    tr = n_pad // 16
    zch = tr // ZR
    ch = e_pad // (2 * 16 * CHUNK)
    mesh = plsc.VectorSubcoreMesh(core_axis_name="c", subcore_axis_name="s")

    @functools.partial(
        pl.kernel,
        mesh=mesh,
        out_type=jax.ShapeDtypeStruct((2, n_pad, 16), F32),
        compiler_params=_SC_PARAMS,
        scratch_types=[
            pltpu.VMEM((CHUNK,), jnp.int32),
            pltpu.VMEM((CHUNK, 16), F32),
            pltpu.VMEM((ZR, 16), F32),
            pltpu.VMEM_SHARED((n_pad, 16), F32),
        ],
    )
    def deg(dst_hbm, out_hbm, didx, ones, zbuf, acc):
        c = lax.axis_index("c")
        t = lax.axis_index("s")
        row0 = t * tr

        def _fill(r, carry):
            zbuf[r, pl.ds(0, 16)] = jnp.zeros((16,), F32)
            return carry

        lax.fori_loop(0, ZR, _fill, 0)

        def _ofill(r, carry):
            ones[r, pl.ds(0, 16)] = jnp.full((16,), 1.0, F32)
            return carry

        lax.fori_loop(0, CHUNK, _ofill, 0)

        for z in range(zch):
            pltpu.sync_copy(zbuf, acc.at[pl.ds(row0 + z * ZR, ZR)])
        plsc.subcore_barrier()

        ebase = c * (e_pad // 2) + t * (ch * CHUNK)

        def _edges(k, carry):
            off = ebase + k * CHUNK
            pltpu.sync_copy(dst_hbm.at[pl.ds(off, CHUNK)], didx)
            pltpu.sync_copy(ones, acc.at[didx], add=True)
            return carry

        lax.fori_loop(0, ch, _edges, 0)
        plsc.subcore_barrier()
        for z in range(zch):
            r0 = row0 + z * ZR
            pltpu.sync_copy(acc.at[pl.ds(r0, ZR)], out_hbm.at[c, pl.ds(r0, ZR)])
        plsc.subcore_barrier()

    return deg


def _rowspec(shape_tail):
    return pl.BlockSpec((R,) + shape_tail, lambda i: (i,) + (0,) * len(shape_tail))


def _uspec():
    return pl.BlockSpec((2, R, 16), lambda i: (0, i, 0))


def _full(shape):
    nd = len(shape)
    return pl.BlockSpec(shape, lambda i: (0,) * nd)


def _sum_u(u_refs, dinv):
    return jnp.concatenate(
        [dinv * (u[0] + u[1]) for u in u_refs], axis=1)


def _dinv_tc(degpart, n_pad):
    def body(d_ref, dinv_ref, dinv2_ref):
        deg = d_ref[0, :, 0:1] + d_ref[1, :, 0:1] + 1.0
        dinv_ref[...] = lax.rsqrt(deg)
        dinv2_ref[...] = 1.0 / deg

    return pl.pallas_call(
        body,
        grid=(n_pad // R,),
        in_specs=[_uspec()],
        out_specs=[_rowspec((1,)), _rowspec((1,))],
        out_shape=[jax.ShapeDtypeStruct((n_pad, 1), F32)] * 2,
    )(degpart)


def _prep0_tc(x_pad, dinv, n_pad):
    def body(x_ref, dinv_ref, s_ref):
        sx = x_ref[...] * dinv_ref[...]
        s_ref[...] = jnp.concatenate([sx, jnp.zeros((R, 13), F32)], axis=1)

    return pl.pallas_call(
        body,
        grid=(n_pad // R,),
        in_specs=[_rowspec((3,)), _rowspec((1,))],
        out_specs=_rowspec((16,)),
        out_shape=jax.ShapeDtypeStruct((n_pad, 16), F32),
    )(x_pad, dinv)


def _layer0_tc(u0, x_pad, dinv, dinv2, W0, b0, n_pad):
    def body(u_ref, x_ref, dinv_ref, dinv2_ref, w_ref, b_ref, h_ref, *s_refs):
        dinv = dinv_ref[...]
        agg = dinv * (u_ref[0][:, 0:3] + u_ref[1][:, 0:3])
        t = agg + dinv2_ref[...] * x_ref[...]
        z = jnp.dot(t, w_ref[...], preferred_element_type=F32) + b_ref[...]
        h = jnp.maximum(z, 0.0)
        h_ref[...] = h
        sh = dinv * h
        for j, s_ref in enumerate(s_refs):
            s_ref[...] = sh[:, 16 * j:16 * (j + 1)]

    return pl.pallas_call(
        body,
        grid=(n_pad // R,),
        in_specs=[_uspec(), _rowspec((3,)), _rowspec((1,)), _rowspec((1,)),
                  _full((3, 64)), _full((1, 64))],
        out_specs=[_rowspec((64,))] + [_rowspec((16,))] * 4,
        out_shape=[jax.ShapeDtypeStruct((n_pad, 64), F32)]
        + [jax.ShapeDtypeStruct((n_pad, 16), F32)] * 4,
    )(u0, x_pad, dinv, dinv2, W0, b0.reshape(1, -1))


def _mm_stats_tc(us, hprev, dinv, dinv2, W, b, cin, cout, n_pad, n_valid):
    grid = n_pad // R
    nb = len(us)

    def body(*refs):
        u_refs = refs[:nb]
        h_ref, dinv_ref, dinv2_ref, w_ref, b_ref, z_ref, st_ref = refs[nb:]
        i = pl.program_id(0)
        t = _sum_u(u_refs, dinv_ref[...]) + dinv2_ref[...] * h_ref[...]
        z = jnp.dot(t, w_ref[...], preferred_element_type=F32) + b_ref[...]
        z_ref[...] = z
        rows = i * R + lax.broadcasted_iota(jnp.int32, (R, 1), 0)
        zm = jnp.where(rows < n_valid, z, 0.0)
        st_ref[0, 0, :] = jnp.sum(zm, axis=0)
        st_ref[0, 1, :] = jnp.sum(zm * zm, axis=0)

    return pl.pallas_call(
        body,
        grid=(grid,),
        in_specs=[_uspec()] * nb
        + [_rowspec((cin,)), _rowspec((1,)), _rowspec((1,)),
           _full((cin, cout)), _full((1, cout))],
        out_specs=[_rowspec((cout,)),
                   pl.BlockSpec((1, 2, cout), lambda i: (i, 0, 0))],
        out_shape=[jax.ShapeDtypeStruct((n_pad, cout), F32),
                   jax.ShapeDtypeStruct((grid, 2, cout), F32)],
    )(*us, hprev, dinv, dinv2, W, b.reshape(1, -1))


def _bn_apply_tc(z, stats, g, be, dinv, c, nbo, n_pad, n_valid, eps=1e-5):
    grid = n_pad // R

    def body(z_ref, st_ref, g_ref, be_ref, dinv_ref, h_ref, *s_refs):
        n = jnp.asarray(n_valid, F32)
        m = jnp.sum(st_ref[:, 0, :], axis=0, keepdims=True) / n
        v = jnp.sum(st_ref[:, 1, :], axis=0, keepdims=True) / n - m * m
        scale = g_ref[...] * lax.rsqrt(v + eps)
        shift = be_ref[...] - m * scale
        y = z_ref[...] * scale + shift
        h = jnp.where(y > 0, y, 0.1 * y)
        h_ref[...] = h
        sh = dinv_ref[...] * h
        for j, s_ref in enumerate(s_refs):
            s_ref[...] = sh[:, 16 * j:16 * (j + 1)]

    return pl.pallas_call(
        body,
        grid=(grid,),
        in_specs=[_rowspec((c,)), _full((grid, 2, c)),
                  _full((1, c)), _full((1, c)), _rowspec((1,))],
        out_specs=[_rowspec((c,))] + [_rowspec((16,))] * nbo,
        out_shape=[jax.ShapeDtypeStruct((n_pad, c), F32)]
        + [jax.ShapeDtypeStruct((n_pad, 16), F32)] * nbo,
    )(z, stats, g.reshape(1, -1), be.reshape(1, -1), dinv)


def _layer_relu_tc(us, hprev, dinv, dinv2, W, b, cin, cout, nbo, n_pad):
    nb = len(us)

    def body(*refs):
        u_refs = refs[:nb]
        h_ref, dinv_ref, dinv2_ref, w_ref, b_ref = refs[nb:nb + 5]
        ho_ref = refs[nb + 5]
        s_refs = refs[nb + 6:]
        dinv = dinv_ref[...]
        t = _sum_u(u_refs, dinv) + dinv2_ref[...] * h_ref[...]
        z = jnp.dot(t, w_ref[...], preferred_element_type=F32) + b_ref[...]
        h = jnp.maximum(z, 0.0)
        ho_ref[...] = h
        sh = dinv * h
        for j, s_ref in enumerate(s_refs):
            s_ref[...] = sh[:, 16 * j:16 * (j + 1)]

    return pl.pallas_call(
        body,
        grid=(n_pad // R,),
        in_specs=[_uspec()] * nb
        + [_rowspec((cin,)), _rowspec((1,)), _rowspec((1,)),
           _full((cin, cout)), _full((1, cout))],
        out_specs=[_rowspec((cout,))] + [_rowspec((16,))] * nbo,
        out_shape=[jax.ShapeDtypeStruct((n_pad, cout), F32)]
        + [jax.ShapeDtypeStruct((n_pad, 16), F32)] * nbo,
    )(*us, hprev, dinv, dinv2, W, b.reshape(1, -1))


def _layer4_tc(us, h3, dinv, dinv2, W4, b4, W5, n_pad):
    def body(*refs):
        u_refs = refs[:16]
        h_ref, dinv_ref, dinv2_ref, w4_ref, b4_ref, w5_ref, v_ref, s_ref = refs[16:]
        dinv = dinv_ref[...]
        t = _sum_u(u_refs, dinv) + dinv2_ref[...] * h_ref[...]
        z = jnp.dot(t, w4_ref[...], preferred_element_type=F32) + b4_ref[...]
        h4 = jnp.maximum(z, 0.0)
        v = jnp.dot(h4, w5_ref[...], preferred_element_type=F32)
        v_ref[...] = v
        s_ref[...] = jnp.concatenate([dinv * v, jnp.zeros((R, 15), F32)], axis=1)

    return pl.pallas_call(
        body,
        grid=(n_pad // R,),
        in_specs=[_uspec()] * 16
        + [_rowspec((256,)), _rowspec((1,)), _rowspec((1,)),
           _full((256, 512)), _full((1, 512)), _full((512, 1))],
        out_specs=[_rowspec((1,)), _rowspec((16,))],
        out_shape=[jax.ShapeDtypeStruct((n_pad, 1), F32),
                   jax.ShapeDtypeStruct((n_pad, 16), F32)],
    )(*us, h3, dinv, dinv2, W4, b4.reshape(1, -1), W5)


def _final_tc(u5, v5, dinv, dinv2, b5, n_pad):
    def body(u_ref, v_ref, dinv_ref, dinv2_ref, b_ref, o_ref):
        agg = dinv_ref[...] * (u_ref[0][:, 0:1] + u_ref[1][:, 0:1])
        o = agg + dinv2_ref[...] * v_ref[...] + b_ref[...]
        o_ref[...] = jax.nn.sigmoid(o)

    return pl.pallas_call(
        body,
        grid=(n_pad // R,),
        in_specs=[_uspec(), _rowspec((1,)), _rowspec((1,)), _rowspec((1,)),
                  _full((1, 1))],
        out_specs=_rowspec((1,)),
        out_shape=jax.ShapeDtypeStruct((n_pad, 1), F32),
    )(u5, v5, dinv, dinv2, b5.reshape(1, 1))


def kernel(x, edge_index, W0, b0, W1, b1, W2, b2, W3, b3, W4, b4, W5, b5,
           g1, be1, g2, be2):
    n = x.shape[0]
    e = edge_index.shape[1]
    n_pad, e_pad = _pads(n, e)

    dummy = jnp.full((e_pad - e,), n_pad - 1, jnp.int32)
    src = jnp.concatenate([edge_index[0], dummy])
    dst = jnp.concatenate([edge_index[1], dummy])
    x_pad = jnp.pad(x, ((0, n_pad - n), (0, 0)))

    agg = _make_agg(n_pad, e_pad)

    degpart = _make_deg(n_pad, e_pad)(dst)
    dinv, dinv2 = _dinv_tc(degpart, n_pad)

    s0 = _prep0_tc(x_pad, dinv, n_pad)
    u0 = agg(s0, src, dst)
    h0, *s1 = _layer0_tc(u0, x_pad, dinv, dinv2, W0, b0, n_pad)

    u1 = [agg(sj, src, dst) for sj in s1]
    z1, st1 = _mm_stats_tc(u1, h0, dinv, dinv2, W1, b1, 64, 64, n_pad, n)
    h1, *s2 = _bn_apply_tc(z1, st1, g1, be1, dinv, 64, 4, n_pad, n)

    u2 = [agg(sj, src, dst) for sj in s2]
    h2, *s3 = _layer_relu_tc(u2, h1, dinv, dinv2, W2, b2, 64, 256, 16, n_pad)

    u3 = [agg(sj, src, dst) for sj in s3]
    z3, st3 = _mm_stats_tc(u3, h2, dinv, dinv2, W3, b3, 256, 256, n_pad, n)
    h3, *s4 = _bn_apply_tc(z3, st3, g2, be2, dinv, 256, 16, n_pad, n)

    u4 = [agg(sj, src, dst) for sj in s4]
    v5, s5 = _layer4_tc(u4, h3, dinv, dinv2, W4, b4, W5, n_pad)

    u5 = agg(s5, src, dst)
    out = _final_tc(u5, v5, dinv, dinv2, b5, n_pad)
    return out[:n]
